# Initial kernel scaffold; baseline (speedup 1.0000x reference)
import functools

import jax
import jax.numpy as jnp
from jax.experimental import pallas as pl
from jax.experimental.pallas import tpu as pltpu
from jax.experimental.pallas import tpu_sc as plsc

N = 10000
E = 320000
D = 128
NC = 2
NS = 16
NW = NC * NS
CHUNK = 128
CPW = 80
NCHUNKS = NW * CPW
EPAD = NCHUNKS * CHUNK
NPAD = 16
NACC = N + NPAD
INIT_ROWS = NACC // NS
OUT_ROWS = N // NS

_MESH = plsc.VectorSubcoreMesh(core_axis_name="c", subcore_axis_name="s")


@functools.partial(
    pl.kernel,
    out_type=jax.ShapeDtypeStruct((NC, N, 16), jnp.float32),
    mesh=_MESH,
    scratch_types=[
        pltpu.VMEM((CPW, CHUNK), jnp.int32),
        pltpu.VMEM((CHUNK, 16), jnp.float32),
        pltpu.VMEM_SHARED((NACC, 16), jnp.float32),
    ],
)
def _deg_kernel(dst_hbm, zeros16_hbm, out_hbm, dst_v, ones_v, acc):
    c = jax.lax.axis_index("c")
    s = jax.lax.axis_index("s")
    w = c * NS + s
    pltpu.sync_copy(zeros16_hbm.at[pl.ds(s * INIT_ROWS, INIT_ROWS)],
                    acc.at[pl.ds(s * INIT_ROWS, INIT_ROWS)])
    pltpu.sync_copy(dst_hbm.at[pl.ds(w * CPW, CPW)], dst_v)

    @pl.loop(0, CHUNK)
    def _(r):
        ones_v[r, :] = jnp.full((16,), 1.0, jnp.float32)

    plsc.subcore_barrier()

    @pl.loop(0, CPW)
    def _(i):
        pltpu.sync_copy(ones_v, acc.at[dst_v.at[i]], add=True)

    plsc.subcore_barrier()
    pltpu.sync_copy(acc.at[pl.ds(s * OUT_ROWS, OUT_ROWS)],
                    out_hbm.at[c, pl.ds(s * OUT_ROWS, OUT_ROWS)])


@functools.partial(
    pl.kernel,
    out_type=jax.ShapeDtypeStruct((NC, N, D), jnp.float32),
    mesh=_MESH,
    scratch_types=[
        pltpu.VMEM((CPW, CHUNK), jnp.int32),
        pltpu.VMEM((CPW, CHUNK), jnp.int32),
        pltpu.VMEM((CHUNK, D), jnp.float32),
        pltpu.VMEM((CHUNK, D), jnp.float32),
        pltpu.VMEM_SHARED((NACC, D), jnp.float32),
        pltpu.SemaphoreType.DMA,
        pltpu.SemaphoreType.DMA,
    ],
)
def _edge_kernel(h_hbm, src_hbm, dst_hbm, zeros_hbm, out_hbm,
                 src_v, dst_v, bufa, bufb, acc, sema, semb):
    c = jax.lax.axis_index("c")
    s = jax.lax.axis_index("s")
    w = c * NS + s
    pltpu.sync_copy(zeros_hbm.at[pl.ds(s * INIT_ROWS, INIT_ROWS)],
                    acc.at[pl.ds(s * INIT_ROWS, INIT_ROWS)])
    pltpu.sync_copy(src_hbm.at[pl.ds(w * CPW, CPW)], src_v)
    pltpu.sync_copy(dst_hbm.at[pl.ds(w * CPW, CPW)], dst_v)
    plsc.subcore_barrier()

    @pl.loop(0, CPW, step=2)
    def _(i):
        cg0 = pltpu.async_copy(h_hbm.at[src_v.at[i]], bufa, sema)
        cg1 = pltpu.async_copy(h_hbm.at[src_v.at[i + 1]], bufb, semb)
        cg0.wait()
        pltpu.sync_copy(bufa, acc.at[dst_v.at[i]], add=True)
        cg1.wait()
        pltpu.sync_copy(bufb, acc.at[dst_v.at[i + 1]], add=True)

    plsc.subcore_barrier()
    pltpu.sync_copy(acc.at[pl.ds(s * OUT_ROWS, OUT_ROWS)],
                    out_hbm.at[c, pl.ds(s * OUT_ROWS, OUT_ROWS)])


_RB = 1000
_HIGH = jax.lax.Precision.HIGHEST


def _dinv_block(d0, d1):
    return jax.lax.rsqrt(1.0 + d0[:, 0:1] + d1[:, 0:1])


def _mm1_body(x_ref, w_ref, d0_ref, d1_ref, h1p_ref):
    dinv = _dinv_block(d0_ref[...], d1_ref[...])
    h = jnp.dot(x_ref[...], w_ref[...],
                preferred_element_type=jnp.float32, precision=_HIGH)
    h1p_ref[...] = h * dinv


def _mid_body(p0_ref, p1_ref, h1p_ref, d0_ref, d1_ref, b_ref, w_ref, out_ref):
    dinv = _dinv_block(d0_ref[...], d1_ref[...])
    t = dinv * (p0_ref[...] + p1_ref[...] + h1p_ref[...]) + b_ref[...]
    r = jnp.maximum(t, 0.0)
    h = jnp.dot(r, w_ref[...],
                preferred_element_type=jnp.float32, precision=_HIGH)
    out_ref[...] = h * dinv


def _final_body(q0_ref, q1_ref, h2p_ref, d0_ref, d1_ref, b_ref, out_ref):
    dinv = _dinv_block(d0_ref[...], d1_ref[...])
    out_ref[...] = dinv * (q0_ref[...] + q1_ref[...] + h2p_ref[...]) + b_ref[...]


def _row_spec(width):
    return pl.BlockSpec((_RB, width), lambda i: (i, 0))


def _full_spec(shape):
    return pl.BlockSpec(shape, lambda i: (0,) * len(shape))


_GRID = (N // _RB,)
_F32_OUT = jax.ShapeDtypeStruct((N, D), jnp.float32)

_mm1 = pl.pallas_call(
    _mm1_body,
    grid=_GRID,
    in_specs=[_row_spec(D), _full_spec((D, D)), _row_spec(16), _row_spec(16)],
    out_specs=_row_spec(D),
    out_shape=_F32_OUT,
)

_mid = pl.pallas_call(
    _mid_body,
    grid=_GRID,
    in_specs=[_row_spec(D), _row_spec(D), _row_spec(D), _row_spec(16),
              _row_spec(16), _full_spec((1, D)), _full_spec((D, D))],
    out_specs=_row_spec(D),
    out_shape=_F32_OUT,
)

_final = pl.pallas_call(
    _final_body,
    grid=_GRID,
    in_specs=[_row_spec(D), _row_spec(D), _row_spec(D), _row_spec(16),
              _row_spec(16), _full_spec((1, D))],
    out_specs=_row_spec(D),
    out_shape=_F32_OUT,
)


def kernel(x, edge_index, W1, b1, W2, b2):
    src = edge_index[0]
    dst = edge_index[1]
    npad = EPAD - E
    pad_pos = jnp.arange(npad, dtype=jnp.int32)
    src_r = jnp.concatenate([src, pad_pos % N]).reshape(NCHUNKS, CHUNK)
    dst_r = jnp.concatenate([dst, N + (pad_pos % NPAD)]).reshape(NCHUNKS, CHUNK)
    zeros = jnp.zeros((NACC, D), jnp.float32)
    zeros16 = jnp.zeros((NACC, 16), jnp.float32)

    degp = _deg_kernel(dst_r, zeros16)
    d0, d1 = degp[0], degp[1]
    h1p = _mm1(x, W1, d0, d1)
    p = _edge_kernel(h1p, src_r, dst_r, zeros)
    h2p = _mid(p[0], p[1], h1p, d0, d1, b1.reshape(1, D), W2)
    q = _edge_kernel(h2p, src_r, dst_r, zeros)
    return _final(q[0], q[1], h2p, d0, d1, b2.reshape(1, D))


# speedup vs baseline: 20.8922x; 20.8922x over previous
import functools

import jax
import jax.numpy as jnp
from jax.experimental import pallas as pl
from jax.experimental.pallas import tpu as pltpu
from jax.experimental.pallas import tpu_sc as plsc

N = 10000
E = 320000
D = 128
NC = 2
NS = 16
NW = NC * NS
CHUNK = 128
CPW = 80
HCPW = 40
NCHUNKS = NW * CPW
EPAD = NCHUNKS * CHUNK
NPAD = 112
NACC = N + NPAD
INIT_ROWS = NACC // NS

_MESH = plsc.VectorSubcoreMesh(core_axis_name="c", subcore_axis_name="s")


@functools.partial(
    pl.kernel,
    out_type=jax.ShapeDtypeStruct((NC, NACC, D), jnp.float32),
    mesh=_MESH,
    scratch_types=[
        pltpu.VMEM((CPW, CHUNK), jnp.int32),
        pltpu.VMEM((CHUNK, D), jnp.float32),
        pltpu.VMEM_SHARED((NACC, D), jnp.float32),
    ],
)
def _deg_kernel(dst_hbm, zeros_hbm, ones_hbm, out_hbm, dst_v, ones_v, acc):
    c = jax.lax.axis_index("c")
    s = jax.lax.axis_index("s")
    w = c * NS + s
    pltpu.sync_copy(zeros_hbm.at[pl.ds(s * INIT_ROWS, INIT_ROWS)],
                    acc.at[pl.ds(s * INIT_ROWS, INIT_ROWS)])
    pltpu.sync_copy(dst_hbm.at[pl.ds(w * CPW, CPW)], dst_v)
    pltpu.sync_copy(ones_hbm, ones_v)
    plsc.subcore_barrier()

    @pl.loop(0, CPW)
    def _(i):
        pltpu.sync_copy(ones_v, acc.at[dst_v.at[i]], add=True)

    plsc.subcore_barrier()
    pltpu.sync_copy(acc.at[pl.ds(s * INIT_ROWS, INIT_ROWS)],
                    out_hbm.at[c, pl.ds(s * INIT_ROWS, INIT_ROWS)])


@functools.partial(
    pl.kernel,
    out_type=jax.ShapeDtypeStruct((NC, NACC, D), jnp.float32),
    mesh=_MESH,
    scratch_types=[
        pltpu.VMEM((HCPW, CHUNK), jnp.int32),
        pltpu.VMEM((HCPW, CHUNK), jnp.int32),
        pltpu.VMEM((CHUNK, D), jnp.float32),
        pltpu.VMEM((CHUNK, D), jnp.float32),
        pltpu.VMEM_SHARED((NACC, D), jnp.float32),
        pltpu.SemaphoreType.DMA,
        pltpu.SemaphoreType.DMA,
    ],
)
def _edge_kernel(h_hbm, src_hbm, dst_hbm, zeros_hbm, out_hbm,
                 src_v, dst_v, bufa, bufb, acc, sema, semb):
    c = jax.lax.axis_index("c")
    s = jax.lax.axis_index("s")
    w = c * NS + s
    pltpu.sync_copy(zeros_hbm.at[pl.ds(s * INIT_ROWS, INIT_ROWS)],
                    acc.at[pl.ds(s * INIT_ROWS, INIT_ROWS)])
    for ph in range(CPW // HCPW):
        pltpu.sync_copy(src_hbm.at[pl.ds(w * CPW + ph * HCPW, HCPW)], src_v)
        pltpu.sync_copy(dst_hbm.at[pl.ds(w * CPW + ph * HCPW, HCPW)], dst_v)
        if ph == 0:
            plsc.subcore_barrier()

        @pl.loop(0, HCPW, step=2)
        def _(i):
            cg0 = pltpu.async_copy(h_hbm.at[src_v.at[i]], bufa, sema)
            cg1 = pltpu.async_copy(h_hbm.at[src_v.at[i + 1]], bufb, semb)
            cg0.wait()
            pltpu.sync_copy(bufa, acc.at[dst_v.at[i]], add=True)
            cg1.wait()
            pltpu.sync_copy(bufb, acc.at[dst_v.at[i + 1]], add=True)

    plsc.subcore_barrier()
    pltpu.sync_copy(acc.at[pl.ds(s * INIT_ROWS, INIT_ROWS)],
                    out_hbm.at[c, pl.ds(s * INIT_ROWS, INIT_ROWS)])


_RB = 1000
_HIGH = jax.lax.Precision.HIGHEST


def _dinv_block(d0, d1):
    return jax.lax.rsqrt(1.0 + d0[:, 0:1] + d1[:, 0:1])


def _mm1_body(x_ref, w_ref, d0_ref, d1_ref, h1p_ref):
    dinv = _dinv_block(d0_ref[...], d1_ref[...])
    h = jnp.dot(x_ref[...], w_ref[...],
                preferred_element_type=jnp.float32, precision=_HIGH)
    h1p_ref[...] = h * dinv


def _mid_body(p0_ref, p1_ref, h1p_ref, d0_ref, d1_ref, b_ref, w_ref, out_ref):
    dinv = _dinv_block(d0_ref[...], d1_ref[...])
    t = dinv * (p0_ref[...] + p1_ref[...] + h1p_ref[...]) + b_ref[...]
    r = jnp.maximum(t, 0.0)
    h = jnp.dot(r, w_ref[...],
                preferred_element_type=jnp.float32, precision=_HIGH)
    out_ref[...] = h * dinv


def _final_body(q0_ref, q1_ref, h2p_ref, d0_ref, d1_ref, b_ref, out_ref):
    dinv = _dinv_block(d0_ref[...], d1_ref[...])
    out_ref[...] = dinv * (q0_ref[...] + q1_ref[...] + h2p_ref[...]) + b_ref[...]


def _row_spec(width):
    return pl.BlockSpec((_RB, width), lambda i: (i, 0))


def _full_spec(shape):
    return pl.BlockSpec(shape, lambda i: (0,) * len(shape))


_GRID = (N // _RB,)
_F32_OUT = jax.ShapeDtypeStruct((N, D), jnp.float32)

_mm1 = pl.pallas_call(
    _mm1_body,
    grid=_GRID,
    in_specs=[_row_spec(D), _full_spec((D, D)), _row_spec(D), _row_spec(D)],
    out_specs=_row_spec(D),
    out_shape=_F32_OUT,
)

_mid = pl.pallas_call(
    _mid_body,
    grid=_GRID,
    in_specs=[_row_spec(D), _row_spec(D), _row_spec(D), _row_spec(D),
              _row_spec(D), _full_spec((1, D)), _full_spec((D, D))],
    out_specs=_row_spec(D),
    out_shape=_F32_OUT,
)

_final = pl.pallas_call(
    _final_body,
    grid=_GRID,
    in_specs=[_row_spec(D), _row_spec(D), _row_spec(D), _row_spec(D),
              _row_spec(D), _full_spec((1, D))],
    out_specs=_row_spec(D),
    out_shape=_F32_OUT,
)


def kernel(x, edge_index, W1, b1, W2, b2):
    src = edge_index[0]
    dst = edge_index[1]
    npad = EPAD - E
    pad_pos = jnp.arange(npad, dtype=jnp.int32)
    src_r = jnp.concatenate([src, pad_pos % N]).reshape(NCHUNKS, CHUNK)
    dst_r = jnp.concatenate([dst, N + (pad_pos % NPAD)]).reshape(NCHUNKS, CHUNK)
    zeros = jnp.zeros((NACC, D), jnp.float32)

    ones = jnp.ones((CHUNK, D), jnp.float32)
    degp = _deg_kernel(dst_r, zeros, ones)
    d0, d1 = degp[0], degp[1]
    h1p = _mm1(x, W1, d0, d1)
    p = _edge_kernel(h1p, src_r, dst_r, zeros)
    h2p = _mid(p[0], p[1], h1p, d0, d1, b1.reshape(1, D), W2)
    q = _edge_kernel(h2p, src_r, dst_r, zeros)
    return _final(q[0], q[1], h2p, d0, d1, b2.reshape(1, D))


# speedup vs baseline: 25.4010x; 1.2158x over previous
import functools

import jax
import jax.numpy as jnp
from jax.experimental import pallas as pl
from jax.experimental.pallas import tpu as pltpu
from jax.experimental.pallas import tpu_sc as plsc

N = 10000
E = 320000
D = 128
NC = 2
NS = 16
NW = NC * NS
CHUNK = 128
CPW = 81
NCHUNKS = NW * CPW
EPAD = NCHUNKS * CHUNK
NPAD = 112
NACC = N + NPAD
INIT_ROWS = NACC // NS

_MESH = plsc.VectorSubcoreMesh(core_axis_name="c", subcore_axis_name="s")


@functools.partial(
    pl.kernel,
    out_type=jax.ShapeDtypeStruct((NC, NACC, D), jnp.float32),
    mesh=_MESH,
    scratch_types=[
        pltpu.VMEM((CPW, 2, CHUNK), jnp.int32),
        pltpu.VMEM((CHUNK, D), jnp.float32),
        pltpu.VMEM_SHARED((NACC, D), jnp.float32),
    ],
)
def _deg_kernel(ei_hbm, zeros_hbm, ones_hbm, out_hbm, ei_v, ones_v, acc):
    c = jax.lax.axis_index("c")
    s = jax.lax.axis_index("s")
    w = c * NS + s
    pltpu.sync_copy(zeros_hbm.at[pl.ds(s * INIT_ROWS, INIT_ROWS)],
                    acc.at[pl.ds(s * INIT_ROWS, INIT_ROWS)])
    pltpu.sync_copy(ei_hbm.at[pl.ds(w * CPW, CPW)], ei_v)
    pltpu.sync_copy(ones_hbm, ones_v)
    plsc.subcore_barrier()

    @pl.loop(0, CPW)
    def _(i):
        pltpu.sync_copy(ones_v, acc.at[ei_v.at[i, 1]], add=True)

    plsc.subcore_barrier()
    pltpu.sync_copy(acc.at[pl.ds(s * INIT_ROWS, INIT_ROWS)],
                    out_hbm.at[c, pl.ds(s * INIT_ROWS, INIT_ROWS)])


@functools.partial(
    pl.kernel,
    out_type=jax.ShapeDtypeStruct((NC, NACC, D), jnp.float32),
    mesh=_MESH,
    scratch_types=[
        pltpu.VMEM((2, CHUNK), jnp.int32),
        pltpu.VMEM((2, CHUNK), jnp.int32),
        pltpu.VMEM((2, CHUNK), jnp.int32),
        pltpu.VMEM((CHUNK, D), jnp.float32),
        pltpu.VMEM((CHUNK, D), jnp.float32),
        pltpu.VMEM((CHUNK, D), jnp.float32),
        pltpu.VMEM_SHARED((NACC, D), jnp.float32),
        pltpu.SemaphoreType.DMA,
        pltpu.SemaphoreType.DMA,
        pltpu.SemaphoreType.DMA,
        pltpu.SemaphoreType.DMA,
        pltpu.SemaphoreType.DMA,
        pltpu.SemaphoreType.DMA,
    ],
)
def _edge_kernel(h_hbm, ei_hbm, zeros_hbm, out_hbm,
                 i0, i1, i2, b0, b1, b2, acc,
                 gs0, gs1, gs2, ss0, ss1, ss2):
    c = jax.lax.axis_index("c")
    s = jax.lax.axis_index("s")
    w = c * NS + s
    base = w * CPW
    ibufs = (i0, i1, i2)
    bufs = (b0, b1, b2)
    gsems = (gs0, gs1, gs2)
    ssems = (ss0, ss1, ss2)

    def wait_scatter(k):
        pltpu.make_async_copy(bufs[k], acc.at[ibufs[k].at[1]], ssems[k]).wait()

    def wait_gather(k):
        pltpu.make_async_copy(h_hbm.at[ibufs[k].at[0]], bufs[k],
                              gsems[k]).wait()

    def start_scatter(k):
        pltpu.async_copy(bufs[k], acc.at[ibufs[k].at[1]], ssems[k], add=True)

    pltpu.sync_copy(zeros_hbm.at[pl.ds(s * INIT_ROWS, INIT_ROWS)],
                    acc.at[pl.ds(s * INIT_ROWS, INIT_ROWS)])
    plsc.subcore_barrier()

    @pl.loop(0, CPW, step=3)
    def _(i):
        for b in range(3):
            j = i + b

            @pl.when(j >= 3)
            def _():
                wait_scatter(b)

            pltpu.sync_copy(ei_hbm.at[base + j], ibufs[b])
            pltpu.async_copy(h_hbm.at[ibufs[b].at[0]], bufs[b], gsems[b])
            k = (b + 1) % 3

            @pl.when(j >= 2)
            def _():
                wait_gather(k)
                start_scatter(k)

    for j in (CPW - 2, CPW - 1):
        wait_gather(j % 3)
        start_scatter(j % 3)
    for b in range(3):
        wait_scatter(b)

    plsc.subcore_barrier()
    pltpu.sync_copy(acc.at[pl.ds(s * INIT_ROWS, INIT_ROWS)],
                    out_hbm.at[c, pl.ds(s * INIT_ROWS, INIT_ROWS)])


_RB = 1000
_HIGH = jax.lax.Precision.HIGHEST


def _dinv_block(d_ref):
    return jax.lax.rsqrt(1.0 + d_ref[0, :, 0:1] + d_ref[1, :, 0:1])


def _mm1_body(x_ref, w_ref, d_ref, h1p_ref):
    h = jnp.dot(x_ref[...], w_ref[...],
                preferred_element_type=jnp.float32, precision=_HIGH)
    h1p_ref[...] = h * _dinv_block(d_ref)


def _mid_body(p_ref, h1p_ref, d_ref, b_ref, w_ref, out_ref):
    dinv = _dinv_block(d_ref)
    t = dinv * (p_ref[0] + p_ref[1] + h1p_ref[...]) + b_ref[...]
    r = jnp.maximum(t, 0.0)
    h = jnp.dot(r, w_ref[...],
                preferred_element_type=jnp.float32, precision=_HIGH)
    out_ref[...] = h * dinv


def _final_body(q_ref, h2p_ref, d_ref, b_ref, out_ref):
    dinv = _dinv_block(d_ref)
    out_ref[...] = dinv * (q_ref[0] + q_ref[1] + h2p_ref[...]) + b_ref[...]


def _row_spec():
    return pl.BlockSpec((_RB, D), lambda i: (i, 0))


def _pair_spec():
    return pl.BlockSpec((NC, _RB, D), lambda i: (0, i, 0))


def _full_spec(shape):
    return pl.BlockSpec(shape, lambda i: (0,) * len(shape))


_GRID = (N // _RB,)
_F32_OUT = jax.ShapeDtypeStruct((N, D), jnp.float32)

_mm1 = pl.pallas_call(
    _mm1_body,
    grid=_GRID,
    in_specs=[_row_spec(), _full_spec((D, D)), _pair_spec()],
    out_specs=_row_spec(),
    out_shape=_F32_OUT,
)

_mid = pl.pallas_call(
    _mid_body,
    grid=_GRID,
    in_specs=[_pair_spec(), _row_spec(), _pair_spec(),
              _full_spec((1, D)), _full_spec((D, D))],
    out_specs=_row_spec(),
    out_shape=_F32_OUT,
)

_final = pl.pallas_call(
    _final_body,
    grid=_GRID,
    in_specs=[_pair_spec(), _row_spec(), _pair_spec(), _full_spec((1, D))],
    out_specs=_row_spec(),
    out_shape=_F32_OUT,
)


def kernel(x, edge_index, W1, b1, W2, b2):
    src = edge_index[0]
    dst = edge_index[1]
    npad = EPAD - E
    pad_pos = jnp.arange(npad, dtype=jnp.int32)
    src_r = jnp.concatenate([src, pad_pos % N]).reshape(NCHUNKS, CHUNK)
    dst_r = jnp.concatenate([dst, N + (pad_pos % NPAD)]).reshape(NCHUNKS, CHUNK)
    ei = jnp.stack([src_r, dst_r], axis=1)
    zeros = jnp.zeros((NACC, D), jnp.float32)
    ones = jnp.ones((CHUNK, D), jnp.float32)

    degp = _deg_kernel(ei, zeros, ones)
    h1p = _mm1(x, W1, degp)
    p = _edge_kernel(h1p, ei, zeros)
    h2p = _mid(p, h1p, degp, b1.reshape(1, D), W2)
    q = _edge_kernel(h2p, ei, zeros)
    return _final(q, h2p, degp, b2.reshape(1, D))


# speedup vs baseline: 28.7847x; 1.1332x over previous
import functools

import jax
import jax.numpy as jnp
from jax.experimental import pallas as pl
from jax.experimental.pallas import tpu as pltpu
from jax.experimental.pallas import tpu_sc as plsc

N = 10000
E = 320000
D = 128
NC = 2
NS = 16
NW = NC * NS
CHUNK = 120
CPW = 84
NCHUNKS = NW * CPW
EPAD = NCHUNKS * CHUNK
NPAD = 112
NACC = N + NPAD
INIT_ROWS = NACC // NS

_MESH = plsc.VectorSubcoreMesh(core_axis_name="c", subcore_axis_name="s")


@functools.partial(
    pl.kernel,
    out_type=jax.ShapeDtypeStruct((NC, NACC, D), jnp.float32),
    mesh=_MESH,
    scratch_types=[
        pltpu.VMEM((CPW, 2, CHUNK), jnp.int32),
        pltpu.VMEM((CHUNK, D), jnp.float32),
        pltpu.VMEM_SHARED((NACC, D), jnp.float32),
    ],
)
def _deg_kernel(ei_hbm, zeros_hbm, ones_hbm, out_hbm, ei_v, ones_v, acc):
    c = jax.lax.axis_index("c")
    s = jax.lax.axis_index("s")
    w = c * NS + s
    pltpu.sync_copy(zeros_hbm.at[pl.ds(s * INIT_ROWS, INIT_ROWS)],
                    acc.at[pl.ds(s * INIT_ROWS, INIT_ROWS)])
    pltpu.sync_copy(ei_hbm.at[pl.ds(w * CPW, CPW)], ei_v)
    pltpu.sync_copy(ones_hbm, ones_v)
    plsc.subcore_barrier()

    @pl.loop(0, CPW)
    def _(i):
        pltpu.sync_copy(ones_v, acc.at[ei_v.at[i, 1]], add=True)

    plsc.subcore_barrier()
    pltpu.sync_copy(acc.at[pl.ds(s * INIT_ROWS, INIT_ROWS)],
                    out_hbm.at[c, pl.ds(s * INIT_ROWS, INIT_ROWS)])


@functools.partial(
    pl.kernel,
    out_type=jax.ShapeDtypeStruct((NC, NACC, D), jnp.float32),
    mesh=_MESH,
    scratch_types=[
        [pltpu.VMEM((2, CHUNK), jnp.int32) for _ in range(6)],
        [pltpu.VMEM((CHUNK, D), jnp.float32) for _ in range(3)],
        pltpu.VMEM_SHARED((NACC, D), jnp.float32),
        [pltpu.SemaphoreType.DMA for _ in range(6)],
        [pltpu.SemaphoreType.DMA for _ in range(3)],
        [pltpu.SemaphoreType.DMA for _ in range(3)],
    ],
)
def _edge_kernel(h_hbm, ei_hbm, zeros_hbm, out_hbm,
                 ibufs, dbufs, acc, isems, gsems, ssems):
    c = jax.lax.axis_index("c")
    s = jax.lax.axis_index("s")
    w = c * NS + s
    base = w * CPW

    def wait_scatter(chunk_mod6, b3):
        pltpu.make_async_copy(dbufs[b3], acc.at[ibufs[chunk_mod6].at[1]],
                              ssems[b3]).wait()

    def wait_gather(chunk_mod6, b3):
        pltpu.make_async_copy(h_hbm.at[ibufs[chunk_mod6].at[0]], dbufs[b3],
                              gsems[b3]).wait()

    def start_scatter(chunk_mod6, b3):
        pltpu.async_copy(dbufs[b3], acc.at[ibufs[chunk_mod6].at[1]],
                         ssems[b3], add=True)

    pltpu.sync_copy(zeros_hbm.at[pl.ds(s * INIT_ROWS, INIT_ROWS)],
                    acc.at[pl.ds(s * INIT_ROWS, INIT_ROWS)])
    for b in range(3):
        pltpu.async_copy(ei_hbm.at[base + b], ibufs[b], isems[b])
    plsc.subcore_barrier()

    @pl.loop(0, CPW, step=6)
    def _(i):
        for b in range(6):
            j = i + b
            b3 = b % 3

            @pl.when(j >= 3)
            def _():
                wait_scatter((b + 3) % 6, b3)

            @pl.when(j + 3 < CPW)
            def _():
                pltpu.async_copy(ei_hbm.at[base + j + 3],
                                 ibufs[(b + 3) % 6], isems[(b + 3) % 6])

            pltpu.make_async_copy(ei_hbm.at[base + j], ibufs[b],
                                  isems[b]).wait()
            pltpu.async_copy(h_hbm.at[ibufs[b].at[0]], dbufs[b3], gsems[b3])

            @pl.when(j >= 2)
            def _():
                wait_gather((b + 4) % 6, (b3 + 1) % 3)
                start_scatter((b + 4) % 6, (b3 + 1) % 3)

    for j in (CPW - 2, CPW - 1):
        wait_gather(j % 6, j % 3)
        start_scatter(j % 6, j % 3)
    for j in (CPW - 3, CPW - 2, CPW - 1):
        wait_scatter(j % 6, j % 3)

    plsc.subcore_barrier()
    pltpu.sync_copy(acc.at[pl.ds(s * INIT_ROWS, INIT_ROWS)],
                    out_hbm.at[c, pl.ds(s * INIT_ROWS, INIT_ROWS)])


_RB = 1000
_HIGH = jax.lax.Precision.HIGHEST


def _dinv_block(d_ref):
    return jax.lax.rsqrt(1.0 + d_ref[0, :, 0:1] + d_ref[1, :, 0:1])


def _mm1_body(x_ref, w_ref, d_ref, h1p_ref):
    h = jnp.dot(x_ref[...], w_ref[...],
                preferred_element_type=jnp.float32, precision=_HIGH)
    h1p_ref[...] = h * _dinv_block(d_ref)


def _mid_body(p_ref, h1p_ref, d_ref, b_ref, w_ref, out_ref):
    dinv = _dinv_block(d_ref)
    t = dinv * (p_ref[0] + p_ref[1] + h1p_ref[...]) + b_ref[...]
    r = jnp.maximum(t, 0.0)
    h = jnp.dot(r, w_ref[...],
                preferred_element_type=jnp.float32, precision=_HIGH)
    out_ref[...] = h * dinv


def _final_body(q_ref, h2p_ref, d_ref, b_ref, out_ref):
    dinv = _dinv_block(d_ref)
    out_ref[...] = dinv * (q_ref[0] + q_ref[1] + h2p_ref[...]) + b_ref[...]


def _row_spec():
    return pl.BlockSpec((_RB, D), lambda i: (i, 0))


def _pair_spec():
    return pl.BlockSpec((NC, _RB, D), lambda i: (0, i, 0))


def _full_spec(shape):
    return pl.BlockSpec(shape, lambda i: (0,) * len(shape))


_GRID = (N // _RB,)
_F32_OUT = jax.ShapeDtypeStruct((N, D), jnp.float32)

_mm1 = pl.pallas_call(
    _mm1_body,
    grid=_GRID,
    in_specs=[_row_spec(), _full_spec((D, D)), _pair_spec()],
    out_specs=_row_spec(),
    out_shape=_F32_OUT,
)

_mid = pl.pallas_call(
    _mid_body,
    grid=_GRID,
    in_specs=[_pair_spec(), _row_spec(), _pair_spec(),
              _full_spec((1, D)), _full_spec((D, D))],
    out_specs=_row_spec(),
    out_shape=_F32_OUT,
)

_final = pl.pallas_call(
    _final_body,
    grid=_GRID,
    in_specs=[_pair_spec(), _row_spec(), _pair_spec(), _full_spec((1, D))],
    out_specs=_row_spec(),
    out_shape=_F32_OUT,
)


def kernel(x, edge_index, W1, b1, W2, b2):
    src = edge_index[0]
    dst = edge_index[1]
    npad = EPAD - E
    pad_pos = jnp.arange(npad, dtype=jnp.int32)
    src_r = jnp.concatenate([src, pad_pos % N]).reshape(NCHUNKS, CHUNK)
    dst_r = jnp.concatenate([dst, N + (pad_pos % NPAD)]).reshape(NCHUNKS, CHUNK)
    ei = jnp.stack([src_r, dst_r], axis=1)
    zeros = jnp.zeros((NACC, D), jnp.float32)
    ones = jnp.ones((CHUNK, D), jnp.float32)

    degp = _deg_kernel(ei, zeros, ones)
    h1p = _mm1(x, W1, degp)
    p = _edge_kernel(h1p, ei, zeros)
    h2p = _mid(p, h1p, degp, b1.reshape(1, D), W2)
    q = _edge_kernel(h2p, ei, zeros)
    return _final(q, h2p, degp, b2.reshape(1, D))


# speedup vs baseline: 33.8816x; 1.1771x over previous
import dataclasses
import functools

import jax
import jax.numpy as jnp
from jax.experimental import pallas as pl
from jax.experimental.pallas import tpu as pltpu
from jax.experimental.pallas import tpu_sc as plsc

N = 10000
E = 320000
D = 128
NC = 2
NS = 16
NW = NC * NS
CHUNK = 120
CPW = 84
NCHUNKS = NW * CPW
EPAD = NCHUNKS * CHUNK
NPAD = 112
NACC = N + NPAD
INIT_ROWS = NACC // NS

_MESH = plsc.VectorSubcoreMesh(core_axis_name="c", subcore_axis_name="s")

_NO_LAYOUT = pltpu.CompilerParams()
if "needs_layout_passes" in pltpu.CompilerParams.__dataclass_fields__:
    _NO_LAYOUT = dataclasses.replace(_NO_LAYOUT, needs_layout_passes=False)


TPW = CPW * CHUNK
NVR = TPW // 16
HB = 80
RPT = HB // NS


@functools.partial(
    pl.kernel,
    out_type=jax.ShapeDtypeStruct((NC, NACC, D), jnp.float32),
    mesh=_MESH,
    compiler_params=_NO_LAYOUT,
    scratch_types=[
        pltpu.VMEM((TPW,), jnp.int32),
        pltpu.VMEM((HB, D), jnp.float32),
        pltpu.VMEM((NS * RPT, D), jnp.float32),
        pltpu.VMEM((D, D), jnp.float32),
        pltpu.VMEM_SHARED((NS, HB, D), jnp.float32),
    ],
)
def _deg_kernel(dst_hbm, zeros_hbm, out_hbm, dst_v, hist, red, piece, hsh):
    c = jax.lax.axis_index("c")
    s = jax.lax.axis_index("s")
    w = c * NS + s
    pltpu.sync_copy(dst_hbm.at[w, 0], dst_v)
    pltpu.sync_copy(zeros_hbm.at[pl.ds(0, HB)], hist)

    @pl.loop(0, NVR)
    def _(i):
        idx = dst_v[pl.ds(i * 16, 16)]
        cnt, last = plsc.scan_count(idx)
        hi = jax.lax.shift_right_logical(idx, 7)
        lo = jax.lax.bitwise_and(idx, 127)
        plsc.addupdate_scatter(hist, [hi, lo], cnt.astype(jnp.float32),
                               mask=last)

    pltpu.sync_copy(hist, hsh.at[s])
    plsc.subcore_barrier()
    for t in range(NS):
        pltpu.sync_copy(hsh.at[t, pl.ds(s * RPT, RPT)],
                        red.at[pl.ds(t * RPT, RPT)])
    lanes = jax.lax.iota(jnp.int32, 16)
    zero16 = jnp.zeros((16,), jnp.int32)
    for r in range(RPT):
        for g in range(D // 16):
            acc = red[r, pl.ds(g * 16, 16)]
            for t in range(1, NS):
                acc = acc + red[t * RPT + r, pl.ds(g * 16, 16)]
            plsc.store_scatter(piece, [lanes + g * 16, zero16], acc)
        row0 = (s * RPT + r) * D

        @pl.when(row0 + D <= NACC)
        def _():
            pltpu.sync_copy(piece, out_hbm.at[c, pl.ds(row0, D)])


@functools.partial(
    pl.kernel,
    out_type=jax.ShapeDtypeStruct((NC, NACC, D), jnp.float32),
    mesh=_MESH,
    scratch_types=[
        [pltpu.VMEM((2, CHUNK), jnp.int32) for _ in range(6)],
        [pltpu.VMEM((CHUNK, D), jnp.float32) for _ in range(3)],
        pltpu.VMEM_SHARED((NACC, D), jnp.float32),
        [pltpu.SemaphoreType.DMA for _ in range(6)],
        [pltpu.SemaphoreType.DMA for _ in range(3)],
        [pltpu.SemaphoreType.DMA for _ in range(3)],
    ],
)
def _edge_kernel(h_hbm, ei_hbm, zeros_hbm, out_hbm,
                 ibufs, dbufs, acc, isems, gsems, ssems):
    c = jax.lax.axis_index("c")
    s = jax.lax.axis_index("s")
    w = c * NS + s
    base = w * CPW

    def wait_scatter(chunk_mod6, b3):
        pltpu.make_async_copy(dbufs[b3], acc.at[ibufs[chunk_mod6].at[1]],
                              ssems[b3]).wait()

    def wait_gather(chunk_mod6, b3):
        pltpu.make_async_copy(h_hbm.at[ibufs[chunk_mod6].at[0]], dbufs[b3],
                              gsems[b3]).wait()

    def start_scatter(chunk_mod6, b3):
        pltpu.async_copy(dbufs[b3], acc.at[ibufs[chunk_mod6].at[1]],
                         ssems[b3], add=True)

    pltpu.sync_copy(zeros_hbm.at[pl.ds(s * INIT_ROWS, INIT_ROWS)],
                    acc.at[pl.ds(s * INIT_ROWS, INIT_ROWS)])
    for b in range(3):
        pltpu.async_copy(ei_hbm.at[base + b], ibufs[b], isems[b])
    plsc.subcore_barrier()

    @pl.loop(0, CPW, step=6)
    def _(i):
        for b in range(6):
            j = i + b
            b3 = b % 3

            @pl.when(j >= 3)
            def _():
                wait_scatter((b + 3) % 6, b3)

            @pl.when(j + 3 < CPW)
            def _():
                pltpu.async_copy(ei_hbm.at[base + j + 3],
                                 ibufs[(b + 3) % 6], isems[(b + 3) % 6])

            pltpu.make_async_copy(ei_hbm.at[base + j], ibufs[b],
                                  isems[b]).wait()
            pltpu.async_copy(h_hbm.at[ibufs[b].at[0]], dbufs[b3], gsems[b3])

            @pl.when(j >= 2)
            def _():
                wait_gather((b + 4) % 6, (b3 + 1) % 3)
                start_scatter((b + 4) % 6, (b3 + 1) % 3)

    for j in (CPW - 2, CPW - 1):
        wait_gather(j % 6, j % 3)
        start_scatter(j % 6, j % 3)
    for j in (CPW - 3, CPW - 2, CPW - 1):
        wait_scatter(j % 6, j % 3)

    plsc.subcore_barrier()
    pltpu.sync_copy(acc.at[pl.ds(s * INIT_ROWS, INIT_ROWS)],
                    out_hbm.at[c, pl.ds(s * INIT_ROWS, INIT_ROWS)])


_RB = 1000
_HIGH = jax.lax.Precision.HIGHEST


def _dinv_block(d_ref):
    return jax.lax.rsqrt(1.0 + d_ref[0, :, 0:1] + d_ref[1, :, 0:1])


def _mm1_body(x_ref, w_ref, d_ref, h1p_ref):
    h = jnp.dot(x_ref[...], w_ref[...],
                preferred_element_type=jnp.float32, precision=_HIGH)
    h1p_ref[...] = h * _dinv_block(d_ref)


def _mid_body(p_ref, h1p_ref, d_ref, b_ref, w_ref, out_ref):
    dinv = _dinv_block(d_ref)
    t = dinv * (p_ref[0] + p_ref[1] + h1p_ref[...]) + b_ref[...]
    r = jnp.maximum(t, 0.0)
    h = jnp.dot(r, w_ref[...],
                preferred_element_type=jnp.float32, precision=_HIGH)
    out_ref[...] = h * dinv


def _final_body(q_ref, h2p_ref, d_ref, b_ref, out_ref):
    dinv = _dinv_block(d_ref)
    out_ref[...] = dinv * (q_ref[0] + q_ref[1] + h2p_ref[...]) + b_ref[...]


def _row_spec():
    return pl.BlockSpec((_RB, D), lambda i: (i, 0))


def _pair_spec():
    return pl.BlockSpec((NC, _RB, D), lambda i: (0, i, 0))


def _full_spec(shape):
    return pl.BlockSpec(shape, lambda i: (0,) * len(shape))


_GRID = (N // _RB,)
_F32_OUT = jax.ShapeDtypeStruct((N, D), jnp.float32)

_mm1 = pl.pallas_call(
    _mm1_body,
    grid=_GRID,
    in_specs=[_row_spec(), _full_spec((D, D)), _pair_spec()],
    out_specs=_row_spec(),
    out_shape=_F32_OUT,
)

_mid = pl.pallas_call(
    _mid_body,
    grid=_GRID,
    in_specs=[_pair_spec(), _row_spec(), _pair_spec(),
              _full_spec((1, D)), _full_spec((D, D))],
    out_specs=_row_spec(),
    out_shape=_F32_OUT,
)

_final = pl.pallas_call(
    _final_body,
    grid=_GRID,
    in_specs=[_pair_spec(), _row_spec(), _pair_spec(), _full_spec((1, D))],
    out_specs=_row_spec(),
    out_shape=_F32_OUT,
)


def kernel(x, edge_index, W1, b1, W2, b2):
    src = edge_index[0]
    dst = edge_index[1]
    npad = EPAD - E
    pad_pos = jnp.arange(npad, dtype=jnp.int32)
    src_r = jnp.concatenate([src, pad_pos % N]).reshape(NCHUNKS, CHUNK)
    dst_r = jnp.concatenate([dst, N + (pad_pos % NPAD)]).reshape(NCHUNKS, CHUNK)
    ei = jnp.stack([src_r, dst_r], axis=1)
    dstw = dst_r.reshape(NW, 1, TPW)
    zeros = jnp.zeros((NACC, D), jnp.float32)

    degp = _deg_kernel(dstw, zeros)
    h1p = _mm1(x, W1, degp)
    p = _edge_kernel(h1p, ei, zeros)
    h2p = _mid(p, h1p, degp, b1.reshape(1, D), W2)
    q = _edge_kernel(h2p, ei, zeros)
    return _final(q, h2p, degp, b2.reshape(1, D))
